# baseline (speedup 1.0000x reference)
import functools

import jax
import jax.numpy as jnp
from jax import lax
from jax.experimental import pallas as pl
from jax.experimental.pallas import tpu as pltpu
from jax.experimental.pallas import tpu_sc as plsc

NC = 2
NS = 16
NW = NC * NS
NBUF = 2


def _embedding_lookup(idx_t, weight, B0, B1, D):
    chunk = B0 // NW
    n_chunks = B1
    mesh = plsc.VectorSubcoreMesh(core_axis_name="c", subcore_axis_name="s")

    @functools.partial(
        pl.kernel,
        out_type=jax.ShapeDtypeStruct((B0 * B1, D), jnp.float32),
        mesh=mesh,
        scratch_types=[
            pltpu.VMEM((n_chunks, chunk), jnp.int32),
            [pltpu.VMEM((chunk, D), jnp.float32) for _ in range(NBUF)],
            [pltpu.SemaphoreType.DMA for _ in range(NBUF)],
            [pltpu.SemaphoreType.DMA for _ in range(NBUF)],
        ],
    )
    def emb(table_hbm, idx_hbm, out_hbm, idx_v, bufs, gsem, wsem):
        wid = lax.axis_index("s") * NC + lax.axis_index("c")
        col0 = wid * chunk
        pltpu.sync_copy(idx_hbm.at[pl.ds(0, 8), pl.ds(col0, chunk)],
                        idx_v.at[pl.ds(0, 8)])

        def start_gather(c, b):
            pltpu.async_copy(table_hbm.at[idx_v.at[c]], bufs[b], gsem[b])

        def wait_gather(c, b):
            pltpu.make_async_copy(table_hbm.at[idx_v.at[c]],
                                  bufs[b], gsem[b]).wait()

        def start_writeback(c, b):
            pltpu.async_copy(
                bufs[b], out_hbm.at[pl.ds(c * B0 + col0, chunk)], wsem[b]
            )

        def wait_writeback(b):
            pltpu.make_async_copy(
                bufs[b], out_hbm.at[pl.ds(col0, chunk)], wsem[b]
            ).wait()

        for b in range(NBUF):
            start_gather(b, b)
        pltpu.sync_copy(
            idx_hbm.at[pl.ds(8, n_chunks - 8), pl.ds(col0, chunk)],
            idx_v.at[pl.ds(8, n_chunks - 8)])

        @pl.loop(0, n_chunks - NBUF, step=NBUF)
        def _(j):
            for b in range(NBUF):
                wait_gather(j + b, b)
                start_writeback(j + b, b)
            for b in range(NBUF):
                wait_writeback(b)
                start_gather(j + b + NBUF, b)

        last = n_chunks - NBUF
        for b in range(NBUF):
            wait_gather(last + b, b)
            start_writeback(last + b, b)
        for b in range(NBUF):
            wait_writeback(b)

    return emb(weight, idx_t)


def kernel(x, weight):
    B0, B1 = x.shape
    V, D = weight.shape
    assert B0 % NW == 0 and B1 % NBUF == 0 and B0 // NW <= 128
    idx_t = x.T.astype(jnp.int32)
    out = _embedding_lookup(idx_t, weight, B0, B1, D)
    return out.reshape(B1, B0, D).transpose(1, 0, 2)
